# baseline (speedup 1.0000x reference)
import functools

import numpy as np

import jax
import jax.numpy as jnp
from jax import lax
from jax.experimental import pallas as pl
from jax.experimental.pallas import tpu as pltpu
from jax.experimental.pallas import tpu_sc as plsc

B, Q, D = 2, 1000, 256
H = 8
LK = 16
Cv = 256
Hs, Ws = 100, 100
SCALE = 0.5

BQ = B * Q
BLK = 200
GRID = BQ // BLK
BPB = Q // BLK

R = H * BQ
NW = 32
RPW = R // NW
OG = 20
NG = RPW // 2
CH = 50


def _sigmoid(x):
    return 1.0 / (1.0 + jnp.exp(-x))


def _stage_a_body(q_ref, g_ref, wa_ref, ba_ref, wo_ref, bo_ref,
                  idx_ref, w_ref, sumw_ref):
    i = pl.program_id(0)
    base = (i // BPB) * (Hs * Ws)

    q = q_ref[...]
    geom = g_ref[...]
    box_x = _sigmoid(geom[:, 0:1])
    box_y = _sigmoid(geom[:, 1:2])
    sx = box_x * (SCALE / LK)
    sy = box_y * (SCALE / LK)

    sumw_cols = []
    for h in range(H):
        logits = jnp.dot(q, wa_ref[h], preferred_element_type=jnp.float32)
        logits = logits + ba_ref[h][None, :]
        m = jnp.max(logits, axis=1, keepdims=True)
        e = jnp.exp(logits - m)
        attn = e / jnp.sum(e, axis=1, keepdims=True)

        off = jnp.dot(q, wo_ref[h], preferred_element_type=jnp.float32)
        off = off + bo_ref[h][None, :]
        locx = box_x + off[:, :LK] * sx
        locy = box_y + off[:, LK:] * sy

        gnx = jnp.clip(2.0 * locx - 1.0, -1.0, 1.0)
        gny = jnp.clip(2.0 * locy - 1.0, -1.0, 1.0)
        gx = ((gnx + 1.0) * Ws - 1.0) * 0.5
        gy = ((gny + 1.0) * Hs - 1.0) * 0.5

        x0f = jnp.floor(gx)
        y0f = jnp.floor(gy)
        wx1 = gx - x0f
        wx0 = 1.0 - wx1
        wy1 = gy - y0f
        wy0 = 1.0 - wy1
        x1f = x0f + 1.0
        y1f = y0f + 1.0

        def corner(xf, yf, wx, wy):
            valid = ((xf >= 0.0) & (xf <= Ws - 1.0)
                     & (yf >= 0.0) & (yf <= Hs - 1.0))
            xi = jnp.clip(xf, 0.0, Ws - 1.0).astype(jnp.int32)
            yi = jnp.clip(yf, 0.0, Hs - 1.0).astype(jnp.int32)
            idx = yi * Ws + xi + base
            w = attn * (wx * wy) * valid.astype(jnp.float32)
            return idx, w

        i00, w00 = corner(x0f, y0f, wx0, wy0)
        i10, w10 = corner(x1f, y0f, wx1, wy0)
        i01, w01 = corner(x0f, y1f, wx0, wy1)
        i11, w11 = corner(x1f, y1f, wx1, wy1)

        idx64 = jnp.concatenate([i00, i10, i01, i11], axis=1)
        w64 = jnp.concatenate([w00, w10, w01, w11], axis=1)
        idx_ref[h] = idx64
        w_ref[h] = w64
        sumw_cols.append(jnp.sum(w64, axis=1, keepdims=True))

    sumw_ref[...] = jnp.concatenate(sumw_cols, axis=1)


def _run_stage_a(q2d, geom, wa, ba, wo, bo):
    return pl.pallas_call(
        _stage_a_body,
        grid=(GRID,),
        in_specs=[
            pl.BlockSpec((BLK, D), lambda i: (i, 0)),
            pl.BlockSpec((BLK, 4), lambda i: (i, 0)),
            pl.BlockSpec((H, D, LK), lambda i: (0, 0, 0)),
            pl.BlockSpec((H, LK), lambda i: (0, 0)),
            pl.BlockSpec((H, D, 2 * LK), lambda i: (0, 0, 0)),
            pl.BlockSpec((H, 2 * LK), lambda i: (0, 0)),
        ],
        out_specs=[
            pl.BlockSpec((H, BLK, 64), lambda i: (0, i, 0)),
            pl.BlockSpec((H, BLK, 64), lambda i: (0, i, 0)),
            pl.BlockSpec((BLK, H), lambda i: (i, 0)),
        ],
        out_shape=[
            jax.ShapeDtypeStruct((H, BQ, 64), jnp.int32),
            jax.ShapeDtypeStruct((H, BQ, 64), jnp.float32),
            jax.ShapeDtypeStruct((BQ, H), jnp.float32),
        ],
    )(q2d, geom, wa, ba, wo, bo)


def _sc_body(table_hbm, idx_hbm, w_hbm, out_hbm, idx_c, w_c, gbuf, obuf, gsem):
    wid = lax.axis_index("s") * 2 + lax.axis_index("c")
    rbase = wid * RPW

    pltpu.sync_copy(idx_hbm.at[pl.ds(rbase * 64, CH * 128)], idx_c.at[0])
    pltpu.sync_copy(w_hbm.at[pl.ds(rbase * 64, CH * 128)], w_c.at[0])
    pltpu.async_copy(table_hbm.at[idx_c.at[0, pl.ds(0, 128)]], gbuf.at[0],
                     gsem.at[0])

    def step(g, carry):
        p = lax.rem(g, 2)
        cslot = lax.rem(g // CH, 2)

        @pl.when(jnp.logical_and(g + 1 < NG, lax.rem(g + 1, CH) == 0))
        def _():
            nslot = lax.rem((g + 1) // CH, 2)
            off = (rbase + (g + 1) * 2) * 64
            pltpu.sync_copy(idx_hbm.at[pl.ds(off, CH * 128)], idx_c.at[nslot])
            pltpu.sync_copy(w_hbm.at[pl.ds(off, CH * 128)], w_c.at[nslot])

        @pl.when(g + 1 < NG)
        def _():
            nslot = lax.rem((g + 1) // CH, 2)
            loff = lax.rem(g + 1, CH) * 128
            pltpu.async_copy(
                table_hbm.at[idx_c.at[nslot, pl.ds(loff, 128)]],
                gbuf.at[1 - p], gsem.at[1 - p])

        pltpu.make_async_copy(
            table_hbm.at[idx_c.at[cslot, pl.ds(lax.rem(g, CH) * 128, 128)]],
            gbuf.at[p], gsem.at[p]).wait()

        for sub in range(2):
            woff = lax.rem(g, CH) * 128 + sub * 64

            def jjbody(jj, acc):
                wvec = w_c[cslot, pl.ds(woff + jj * 16, 16)]
                for l in range(16):
                    j = sub * 64 + jj * 16 + l
                    ws = wvec[l]
                    acc = tuple(acc[c] + gbuf[p, j, pl.ds(c * 16, 16)] * ws
                                for c in range(16))
                return acc

            acc = lax.fori_loop(
                0, 4, jjbody,
                tuple(jnp.zeros((16,), jnp.float32) for _ in range(16)))

            r = lax.rem(2 * g + sub, OG)
            for c in range(16):
                obuf[pl.ds(r * Cv + c * 16, 16)] = acc[c]

            @pl.when(r == OG - 1)
            def _():
                pltpu.sync_copy(
                    obuf,
                    out_hbm.at[pl.ds((rbase + 2 * g + sub + 1 - OG) * Cv,
                                     OG * Cv)])

        return carry

    lax.fori_loop(0, NG, step, 0)


def _run_stage_b(table, idx_flat, w_flat):
    mesh = plsc.VectorSubcoreMesh(core_axis_name="c", subcore_axis_name="s")
    f = functools.partial(
        pl.kernel,
        out_type=jax.ShapeDtypeStruct((R * Cv,), jnp.float32),
        mesh=mesh,
        scratch_types=[
            pltpu.VMEM((2, CH * 128), jnp.int32),
            pltpu.VMEM((2, CH * 128), jnp.float32),
            pltpu.VMEM((2, 128, Cv), jnp.float32),
            pltpu.VMEM((OG * Cv,), jnp.float32),
            pltpu.SemaphoreType.DMA((2,)),
        ],
    )(_sc_body)
    return f(table, idx_flat, w_flat)


def _precompute_body(wv_ref, wf_ref, bv_ref, wc_ref, bb_ref):
    wv = wv_ref[...]
    bv = bv_ref[...]
    bb_rows = []
    for h in range(H):
        wf = wf_ref[h]
        wc_ref[h] = jnp.dot(wv, wf, preferred_element_type=jnp.float32)
        bb_rows.append(jnp.dot(bv, wf, preferred_element_type=jnp.float32))
    bb_ref[...] = jnp.concatenate(bb_rows, axis=0)


def _run_precompute(W_val, W_final, b_val_2d):
    return pl.pallas_call(
        _precompute_body,
        out_shape=[
            jax.ShapeDtypeStruct((H, Cv, Cv), jnp.float32),
            jax.ShapeDtypeStruct((H, Cv), jnp.float32),
        ],
    )(W_val, W_final, b_val_2d)


def _stage_c_body(heads_ref, sumw_ref, wc_ref, bb_ref, bf_ref, out_ref):
    acc = jnp.dot(sumw_ref[...], bb_ref[...],
                  preferred_element_type=jnp.float32)
    for h in range(H):
        acc = acc + jnp.dot(heads_ref[h], wc_ref[h],
                            preferred_element_type=jnp.float32)
    acc = acc + jnp.sum(bf_ref[...], axis=0)[None, :]
    out_ref[...] = acc


def _run_stage_c(heads3, sumw, wc, bb, b_final):
    return pl.pallas_call(
        _stage_c_body,
        grid=(GRID,),
        in_specs=[
            pl.BlockSpec((H, BLK, Cv), lambda i: (0, i, 0)),
            pl.BlockSpec((BLK, H), lambda i: (i, 0)),
            pl.BlockSpec((H, Cv, Cv), lambda i: (0, 0, 0)),
            pl.BlockSpec((H, Cv), lambda i: (0, 0)),
            pl.BlockSpec((H, Cv), lambda i: (0, 0)),
        ],
        out_specs=pl.BlockSpec((BLK, Cv), lambda i: (i, 0)),
        out_shape=jax.ShapeDtypeStruct((BQ, Cv), jnp.float32),
    )(heads3, sumw, wc, bb, b_final)


def kernel(queries, query_geometry_logits, value, W_off, b_off, W_attn,
           b_attn, W_val, b_val, W_final, b_final):
    q2d = queries.reshape(BQ, D)
    geom = query_geometry_logits.reshape(BQ, 4)

    wo = W_off.reshape(D, H, LK, 2).transpose(1, 0, 3, 2).reshape(H, D, 2 * LK)
    bo = b_off.reshape(H, LK, 2).transpose(0, 2, 1).reshape(H, 2 * LK)
    wa = W_attn.reshape(D, H, LK).transpose(1, 0, 2)
    ba = b_attn.reshape(H, LK)

    idx8, w8, sumw = _run_stage_a(q2d, geom, wa, ba, wo, bo)

    table = jnp.transpose(value, (0, 2, 3, 1)).reshape(B * Hs * Ws, Cv)
    heads = _run_stage_b(table, idx8.reshape(R * 64), w8.reshape(R * 64))

    wc, bb = _run_precompute(W_val, W_final, b_val.reshape(1, Cv))
    out2d = _run_stage_c(heads.reshape(H, BQ, Cv), sumw, wc, bb, b_final)
    return out2d.reshape(B, Q, Cv)


# speedup vs baseline: 1.2620x; 1.2620x over previous
import functools

import numpy as np

import jax
import jax.numpy as jnp
from jax import lax
from jax.experimental import pallas as pl
from jax.experimental.pallas import tpu as pltpu
from jax.experimental.pallas import tpu_sc as plsc

B, Q, D = 2, 1000, 256
H = 8
LK = 16
Cv = 256
Hs, Ws = 100, 100
SCALE = 0.5

BQ = B * Q
BLK = 200
GRID = BQ // BLK
BPB = Q // BLK

R = H * BQ
NW = 32
RPW = R // NW
OG = 20
KR = 4
CH = 100


def _sigmoid(x):
    return 1.0 / (1.0 + jnp.exp(-x))


def _stage_a_body(q_ref, g_ref, wa_ref, ba_ref, wo_ref, bo_ref,
                  idx_ref, w_ref, sumw_ref):
    i = pl.program_id(0)
    base = (i // BPB) * (Hs * Ws)

    q = q_ref[...]
    geom = g_ref[...]
    box_x = _sigmoid(geom[:, 0:1])
    box_y = _sigmoid(geom[:, 1:2])
    sx = box_x * (SCALE / LK)
    sy = box_y * (SCALE / LK)

    sumw_cols = []
    for h in range(H):
        logits = jnp.dot(q, wa_ref[h], preferred_element_type=jnp.float32)
        logits = logits + ba_ref[h][None, :]
        m = jnp.max(logits, axis=1, keepdims=True)
        e = jnp.exp(logits - m)
        attn = e / jnp.sum(e, axis=1, keepdims=True)

        off = jnp.dot(q, wo_ref[h], preferred_element_type=jnp.float32)
        off = off + bo_ref[h][None, :]
        locx = box_x + off[:, :LK] * sx
        locy = box_y + off[:, LK:] * sy

        gnx = jnp.clip(2.0 * locx - 1.0, -1.0, 1.0)
        gny = jnp.clip(2.0 * locy - 1.0, -1.0, 1.0)
        gx = ((gnx + 1.0) * Ws - 1.0) * 0.5
        gy = ((gny + 1.0) * Hs - 1.0) * 0.5

        x0f = jnp.floor(gx)
        y0f = jnp.floor(gy)
        wx1 = gx - x0f
        wx0 = 1.0 - wx1
        wy1 = gy - y0f
        wy0 = 1.0 - wy1
        x1f = x0f + 1.0
        y1f = y0f + 1.0

        def corner(xf, yf, wx, wy):
            valid = ((xf >= 0.0) & (xf <= Ws - 1.0)
                     & (yf >= 0.0) & (yf <= Hs - 1.0))
            xi = jnp.clip(xf, 0.0, Ws - 1.0).astype(jnp.int32)
            yi = jnp.clip(yf, 0.0, Hs - 1.0).astype(jnp.int32)
            idx = yi * Ws + xi + base
            w = attn * (wx * wy) * valid.astype(jnp.float32)
            return idx, w

        i00, w00 = corner(x0f, y0f, wx0, wy0)
        i10, w10 = corner(x1f, y0f, wx1, wy0)
        i01, w01 = corner(x0f, y1f, wx0, wy1)
        i11, w11 = corner(x1f, y1f, wx1, wy1)

        idx64 = jnp.concatenate([i00, i10, i01, i11], axis=1)
        w64 = jnp.concatenate([w00, w10, w01, w11], axis=1)
        idx_ref[h] = idx64
        w_ref[h] = w64
        sumw_cols.append(jnp.sum(w64, axis=1, keepdims=True))

    sumw_ref[...] = jnp.concatenate(sumw_cols, axis=1)


def _run_stage_a(q2d, geom, wa, ba, wo, bo):
    return pl.pallas_call(
        _stage_a_body,
        grid=(GRID,),
        in_specs=[
            pl.BlockSpec((BLK, D), lambda i: (i, 0)),
            pl.BlockSpec((BLK, 4), lambda i: (i, 0)),
            pl.BlockSpec((H, D, LK), lambda i: (0, 0, 0)),
            pl.BlockSpec((H, LK), lambda i: (0, 0)),
            pl.BlockSpec((H, D, 2 * LK), lambda i: (0, 0, 0)),
            pl.BlockSpec((H, 2 * LK), lambda i: (0, 0)),
        ],
        out_specs=[
            pl.BlockSpec((H, BLK, 64), lambda i: (0, i, 0)),
            pl.BlockSpec((H, BLK, 64), lambda i: (0, i, 0)),
            pl.BlockSpec((BLK, H), lambda i: (i, 0)),
        ],
        out_shape=[
            jax.ShapeDtypeStruct((H, BQ, 64), jnp.int32),
            jax.ShapeDtypeStruct((H, BQ, 64), jnp.float32),
            jax.ShapeDtypeStruct((BQ, H), jnp.float32),
        ],
    )(q2d, geom, wa, ba, wo, bo)


def _sc_body(table_hbm, idx_hbm, w_hbm, out_hbm, idx_c, w_c, gbuf, obuf, gsem):
    wid = lax.axis_index("s") * 2 + lax.axis_index("c")
    rbase = wid * RPW

    pltpu.sync_copy(idx_hbm.at[pl.ds(rbase * 64, CH * 64)],
                    idx_c.at[pl.ds(0, CH * 64)])
    pltpu.sync_copy(w_hbm.at[pl.ds(rbase * 64, CH * 64)],
                    w_c.at[pl.ds(0, CH * 64)])
    for k in range(KR - 1):
        pltpu.async_copy(table_hbm.at[idx_c.at[pl.ds(k * 64, 64)]],
                         gbuf.at[k], gsem.at[k])

    def step(i, carry):
        p = lax.rem(i, KR)
        cslot = lax.rem(i // CH, 2)
        nxt = i + KR - 1

        @pl.when(jnp.logical_and(nxt < RPW, lax.rem(nxt, CH) == 0))
        def _():
            nbase = lax.rem(nxt // CH, 2) * (CH * 64)
            off = (rbase + nxt) * 64
            pltpu.sync_copy(idx_hbm.at[pl.ds(off, CH * 64)],
                            idx_c.at[pl.ds(nbase, CH * 64)])
            pltpu.sync_copy(w_hbm.at[pl.ds(off, CH * 64)],
                            w_c.at[pl.ds(nbase, CH * 64)])

        @pl.when(nxt < RPW)
        def _():
            nbase = lax.rem(nxt // CH, 2) * (CH * 64)
            loff = nbase + lax.rem(nxt, CH) * 64
            pltpu.async_copy(table_hbm.at[idx_c.at[pl.ds(loff, 64)]],
                             gbuf.at[lax.rem(nxt, KR)],
                             gsem.at[lax.rem(nxt, KR)])

        pltpu.make_async_copy(
            table_hbm.at[idx_c.at[pl.ds(cslot * (CH * 64)
                                        + lax.rem(i, CH) * 64, 64)]],
            gbuf.at[p], gsem.at[p]).wait()

        woff = cslot * (CH * 64) + lax.rem(i, CH) * 64

        def jjbody(jj, acc):
            wvec = w_c[pl.ds(woff + jj * 16, 16)]
            for l in range(16):
                j = jj * 16 + l
                ws = wvec[l]
                acc = tuple(acc[c] + gbuf[p, j, pl.ds(c * 16, 16)] * ws
                            for c in range(16))
            return acc

        acc = lax.fori_loop(
            0, 4, jjbody,
            tuple(jnp.zeros((16,), jnp.float32) for _ in range(16)))

        r = lax.rem(i, OG)
        for c in range(16):
            obuf[pl.ds(r * Cv + c * 16, 16)] = acc[c]

        @pl.when(r == OG - 1)
        def _():
            pltpu.sync_copy(
                obuf, out_hbm.at[pl.ds((rbase + i + 1 - OG) * Cv, OG * Cv)])

        return carry

    lax.fori_loop(0, RPW, step, 0)


def _run_stage_b(table, idx_flat, w_flat):
    mesh = plsc.VectorSubcoreMesh(core_axis_name="c", subcore_axis_name="s")
    f = functools.partial(
        pl.kernel,
        out_type=jax.ShapeDtypeStruct((R * Cv,), jnp.float32),
        mesh=mesh,
        scratch_types=[
            pltpu.VMEM((2 * CH * 64,), jnp.int32),
            pltpu.VMEM((2 * CH * 64,), jnp.float32),
            pltpu.VMEM((KR, 64, Cv), jnp.float32),
            pltpu.VMEM((OG * Cv,), jnp.float32),
            pltpu.SemaphoreType.DMA((KR,)),
        ],
    )(_sc_body)
    return f(table, idx_flat, w_flat)


def _precompute_body(wv_ref, wf_ref, bv_ref, wc_ref, bb_ref):
    wv = wv_ref[...]
    bv = bv_ref[...]
    bb_rows = []
    for h in range(H):
        wf = wf_ref[h]
        wc_ref[h] = jnp.dot(wv, wf, preferred_element_type=jnp.float32)
        bb_rows.append(jnp.dot(bv, wf, preferred_element_type=jnp.float32))
    bb_ref[...] = jnp.concatenate(bb_rows, axis=0)


def _run_precompute(W_val, W_final, b_val_2d):
    return pl.pallas_call(
        _precompute_body,
        out_shape=[
            jax.ShapeDtypeStruct((H, Cv, Cv), jnp.float32),
            jax.ShapeDtypeStruct((H, Cv), jnp.float32),
        ],
    )(W_val, W_final, b_val_2d)


def _stage_c_body(heads_ref, sumw_ref, wc_ref, bb_ref, bf_ref, out_ref):
    acc = jnp.dot(sumw_ref[...], bb_ref[...],
                  preferred_element_type=jnp.float32)
    for h in range(H):
        acc = acc + jnp.dot(heads_ref[h], wc_ref[h],
                            preferred_element_type=jnp.float32)
    acc = acc + jnp.sum(bf_ref[...], axis=0)[None, :]
    out_ref[...] = acc


def _run_stage_c(heads3, sumw, wc, bb, b_final):
    return pl.pallas_call(
        _stage_c_body,
        grid=(GRID,),
        in_specs=[
            pl.BlockSpec((H, BLK, Cv), lambda i: (0, i, 0)),
            pl.BlockSpec((BLK, H), lambda i: (i, 0)),
            pl.BlockSpec((H, Cv, Cv), lambda i: (0, 0, 0)),
            pl.BlockSpec((H, Cv), lambda i: (0, 0)),
            pl.BlockSpec((H, Cv), lambda i: (0, 0)),
        ],
        out_specs=pl.BlockSpec((BLK, Cv), lambda i: (i, 0)),
        out_shape=jax.ShapeDtypeStruct((BQ, Cv), jnp.float32),
    )(heads3, sumw, wc, bb, b_final)


def kernel(queries, query_geometry_logits, value, W_off, b_off, W_attn,
           b_attn, W_val, b_val, W_final, b_final):
    q2d = queries.reshape(BQ, D)
    geom = query_geometry_logits.reshape(BQ, 4)

    wo = W_off.reshape(D, H, LK, 2).transpose(1, 0, 3, 2).reshape(H, D, 2 * LK)
    bo = b_off.reshape(H, LK, 2).transpose(0, 2, 1).reshape(H, 2 * LK)
    wa = W_attn.reshape(D, H, LK).transpose(1, 0, 2)
    ba = b_attn.reshape(H, LK)

    idx8, w8, sumw = _run_stage_a(q2d, geom, wa, ba, wo, bo)

    table = jnp.transpose(value, (0, 2, 3, 1)).reshape(B * Hs * Ws, Cv)
    heads = _run_stage_b(table, idx8.reshape(R * 64), w8.reshape(R * 64))

    wc, bb = _run_precompute(W_val, W_final, b_val.reshape(1, Cv))
    out2d = _run_stage_c(heads.reshape(H, BQ, Cv), sumw, wc, bb, b_final)
    return out2d.reshape(B, Q, Cv)


# speedup vs baseline: 1.2788x; 1.0133x over previous
import functools

import numpy as np

import jax
import jax.numpy as jnp
from jax import lax
from jax.experimental import pallas as pl
from jax.experimental.pallas import tpu as pltpu
from jax.experimental.pallas import tpu_sc as plsc

B, Q, D = 2, 1000, 256
H = 8
LK = 16
Cv = 256
Hs, Ws = 100, 100
SCALE = 0.5

BQ = B * Q
BLK = 200
GRID = BQ // BLK
BPB = Q // BLK

R = H * BQ
NW = 32
RPW = R // NW
OG = 20
KR = 5
CH = 100


def _sigmoid(x):
    return 1.0 / (1.0 + jnp.exp(-x))


def _stage_a_body(q_ref, g_ref, wa_ref, ba_ref, wo_ref, bo_ref,
                  idx_ref, w_ref, sumw_ref):
    i = pl.program_id(0)
    base = (i // BPB) * (Hs * Ws)

    q = q_ref[...]
    geom = g_ref[...]
    box_x = _sigmoid(geom[:, 0:1])
    box_y = _sigmoid(geom[:, 1:2])
    sx = box_x * (SCALE / LK)
    sy = box_y * (SCALE / LK)

    sumw_cols = []
    for h in range(H):
        logits = jnp.dot(q, wa_ref[h], preferred_element_type=jnp.float32)
        logits = logits + ba_ref[h][None, :]
        m = jnp.max(logits, axis=1, keepdims=True)
        e = jnp.exp(logits - m)
        attn = e / jnp.sum(e, axis=1, keepdims=True)

        off = jnp.dot(q, wo_ref[h], preferred_element_type=jnp.float32)
        off = off + bo_ref[h][None, :]
        locx = box_x + off[:, :LK] * sx
        locy = box_y + off[:, LK:] * sy

        gnx = jnp.clip(2.0 * locx - 1.0, -1.0, 1.0)
        gny = jnp.clip(2.0 * locy - 1.0, -1.0, 1.0)
        gx = ((gnx + 1.0) * Ws - 1.0) * 0.5
        gy = ((gny + 1.0) * Hs - 1.0) * 0.5

        x0f = jnp.floor(gx)
        y0f = jnp.floor(gy)
        wx1 = gx - x0f
        wx0 = 1.0 - wx1
        wy1 = gy - y0f
        wy0 = 1.0 - wy1
        x1f = x0f + 1.0
        y1f = y0f + 1.0

        def corner(xf, yf, wx, wy):
            valid = ((xf >= 0.0) & (xf <= Ws - 1.0)
                     & (yf >= 0.0) & (yf <= Hs - 1.0))
            xi = jnp.clip(xf, 0.0, Ws - 1.0).astype(jnp.int32)
            yi = jnp.clip(yf, 0.0, Hs - 1.0).astype(jnp.int32)
            idx = yi * Ws + xi + base
            w = attn * (wx * wy) * valid.astype(jnp.float32)
            return idx, w

        i00, w00 = corner(x0f, y0f, wx0, wy0)
        i10, w10 = corner(x1f, y0f, wx1, wy0)
        i01, w01 = corner(x0f, y1f, wx0, wy1)
        i11, w11 = corner(x1f, y1f, wx1, wy1)

        idx64 = jnp.concatenate([i00, i10, i01, i11], axis=1)
        w64 = jnp.concatenate([w00, w10, w01, w11], axis=1)
        idx_ref[h] = idx64
        w_ref[h] = w64
        sumw_cols.append(jnp.sum(w64, axis=1, keepdims=True))

    sumw_ref[...] = jnp.concatenate(sumw_cols, axis=1)


def _run_stage_a(q2d, geom, wa, ba, wo, bo):
    return pl.pallas_call(
        _stage_a_body,
        grid=(GRID,),
        in_specs=[
            pl.BlockSpec((BLK, D), lambda i: (i, 0)),
            pl.BlockSpec((BLK, 4), lambda i: (i, 0)),
            pl.BlockSpec((H, D, LK), lambda i: (0, 0, 0)),
            pl.BlockSpec((H, LK), lambda i: (0, 0)),
            pl.BlockSpec((H, D, 2 * LK), lambda i: (0, 0, 0)),
            pl.BlockSpec((H, 2 * LK), lambda i: (0, 0)),
        ],
        out_specs=[
            pl.BlockSpec((H, BLK, 64), lambda i: (0, i, 0)),
            pl.BlockSpec((H, BLK, 64), lambda i: (0, i, 0)),
            pl.BlockSpec((BLK, H), lambda i: (i, 0)),
        ],
        out_shape=[
            jax.ShapeDtypeStruct((H, BQ, 64), jnp.int32),
            jax.ShapeDtypeStruct((H, BQ, 64), jnp.float32),
            jax.ShapeDtypeStruct((BQ, H), jnp.float32),
        ],
    )(q2d, geom, wa, ba, wo, bo)


def _sc_body(table_hbm, idx_hbm, w_hbm, out_hbm, idx_c, w_c, gbuf, obuf, gsem):
    wid = lax.axis_index("s") * 2 + lax.axis_index("c")
    rbase = wid * RPW

    pltpu.sync_copy(idx_hbm.at[pl.ds(rbase * 64, CH * 64)],
                    idx_c.at[pl.ds(0, CH * 64)])
    pltpu.sync_copy(w_hbm.at[pl.ds(rbase * 64, CH * 64)],
                    w_c.at[pl.ds(0, CH * 64)])
    for k in range(KR - 1):
        pltpu.async_copy(table_hbm.at[idx_c.at[pl.ds(k * 64, 64)]],
                         gbuf.at[k], gsem.at[k])

    def step(i, carry):
        p = lax.rem(i, KR)
        cslot = lax.rem(i // CH, 2)
        nxt = i + KR - 1

        @pl.when(jnp.logical_and(nxt < RPW, lax.rem(nxt, CH) == 0))
        def _():
            nbase = lax.rem(nxt // CH, 2) * (CH * 64)
            off = (rbase + nxt) * 64
            pltpu.sync_copy(idx_hbm.at[pl.ds(off, CH * 64)],
                            idx_c.at[pl.ds(nbase, CH * 64)])
            pltpu.sync_copy(w_hbm.at[pl.ds(off, CH * 64)],
                            w_c.at[pl.ds(nbase, CH * 64)])

        @pl.when(nxt < RPW)
        def _():
            nbase = lax.rem(nxt // CH, 2) * (CH * 64)
            loff = nbase + lax.rem(nxt, CH) * 64
            pltpu.async_copy(table_hbm.at[idx_c.at[pl.ds(loff, 64)]],
                             gbuf.at[lax.rem(nxt, KR)],
                             gsem.at[lax.rem(nxt, KR)])

        pltpu.make_async_copy(
            table_hbm.at[idx_c.at[pl.ds(cslot * (CH * 64)
                                        + lax.rem(i, CH) * 64, 64)]],
            gbuf.at[p], gsem.at[p]).wait()

        woff = cslot * (CH * 64) + lax.rem(i, CH) * 64

        def jjbody(jj, acc):
            wvec = w_c[pl.ds(woff + jj * 16, 16)]
            for l in range(16):
                j = jj * 16 + l
                ws = wvec[l]
                acc = tuple(acc[c] + gbuf[p, j, pl.ds(c * 16, 16)] * ws
                            for c in range(16))
            return acc

        acc = lax.fori_loop(
            0, 4, jjbody,
            tuple(jnp.zeros((16,), jnp.float32) for _ in range(16)))

        r = lax.rem(i, OG)
        for c in range(16):
            obuf[pl.ds(r * Cv + c * 16, 16)] = acc[c]

        @pl.when(r == OG - 1)
        def _():
            pltpu.sync_copy(
                obuf, out_hbm.at[pl.ds((rbase + i + 1 - OG) * Cv, OG * Cv)])

        return carry

    lax.fori_loop(0, RPW, step, 0)


def _run_stage_b(table, idx_flat, w_flat):
    mesh = plsc.VectorSubcoreMesh(core_axis_name="c", subcore_axis_name="s")
    f = functools.partial(
        pl.kernel,
        out_type=jax.ShapeDtypeStruct((R * Cv,), jnp.float32),
        mesh=mesh,
        scratch_types=[
            pltpu.VMEM((2 * CH * 64,), jnp.int32),
            pltpu.VMEM((2 * CH * 64,), jnp.float32),
            pltpu.VMEM((KR, 64, Cv), jnp.float32),
            pltpu.VMEM((OG * Cv,), jnp.float32),
            pltpu.SemaphoreType.DMA((KR,)),
        ],
    )(_sc_body)
    return f(table, idx_flat, w_flat)


def _precompute_body(wv_ref, wf_ref, bv_ref, wc_ref, bb_ref):
    wv = wv_ref[...]
    bv = bv_ref[...]
    bb_rows = []
    for h in range(H):
        wf = wf_ref[h]
        wc_ref[h] = jnp.dot(wv, wf, preferred_element_type=jnp.float32)
        bb_rows.append(jnp.dot(bv, wf, preferred_element_type=jnp.float32))
    bb_ref[...] = jnp.concatenate(bb_rows, axis=0)


def _run_precompute(W_val, W_final, b_val_2d):
    return pl.pallas_call(
        _precompute_body,
        out_shape=[
            jax.ShapeDtypeStruct((H, Cv, Cv), jnp.float32),
            jax.ShapeDtypeStruct((H, Cv), jnp.float32),
        ],
    )(W_val, W_final, b_val_2d)


def _stage_c_body(heads_ref, sumw_ref, wc_ref, bb_ref, bf_ref, out_ref):
    acc = jnp.dot(sumw_ref[...], bb_ref[...],
                  preferred_element_type=jnp.float32)
    for h in range(H):
        acc = acc + jnp.dot(heads_ref[h], wc_ref[h],
                            preferred_element_type=jnp.float32)
    acc = acc + jnp.sum(bf_ref[...], axis=0)[None, :]
    out_ref[...] = acc


def _run_stage_c(heads3, sumw, wc, bb, b_final):
    return pl.pallas_call(
        _stage_c_body,
        grid=(GRID,),
        in_specs=[
            pl.BlockSpec((H, BLK, Cv), lambda i: (0, i, 0)),
            pl.BlockSpec((BLK, H), lambda i: (i, 0)),
            pl.BlockSpec((H, Cv, Cv), lambda i: (0, 0, 0)),
            pl.BlockSpec((H, Cv), lambda i: (0, 0)),
            pl.BlockSpec((H, Cv), lambda i: (0, 0)),
        ],
        out_specs=pl.BlockSpec((BLK, Cv), lambda i: (i, 0)),
        out_shape=jax.ShapeDtypeStruct((BQ, Cv), jnp.float32),
    )(heads3, sumw, wc, bb, b_final)


def kernel(queries, query_geometry_logits, value, W_off, b_off, W_attn,
           b_attn, W_val, b_val, W_final, b_final):
    q2d = queries.reshape(BQ, D)
    geom = query_geometry_logits.reshape(BQ, 4)

    wo = W_off.reshape(D, H, LK, 2).transpose(1, 0, 3, 2).reshape(H, D, 2 * LK)
    bo = b_off.reshape(H, LK, 2).transpose(0, 2, 1).reshape(H, 2 * LK)
    wa = W_attn.reshape(D, H, LK).transpose(1, 0, 2)
    ba = b_attn.reshape(H, LK)

    idx8, w8, sumw = _run_stage_a(q2d, geom, wa, ba, wo, bo)

    table = jnp.transpose(value, (0, 2, 3, 1)).reshape(B * Hs * Ws, Cv)
    heads = _run_stage_b(table, idx8.reshape(R * 64), w8.reshape(R * 64))

    wc, bb = _run_precompute(W_val, W_final, b_val.reshape(1, Cv))
    out2d = _run_stage_c(heads.reshape(H, BQ, Cv), sumw, wc, bb, b_final)
    return out2d.reshape(B, Q, Cv)


# speedup vs baseline: 1.5246x; 1.1922x over previous
import functools

import numpy as np

import jax
import jax.numpy as jnp
from jax import lax
from jax.experimental import pallas as pl
from jax.experimental.pallas import tpu as pltpu
from jax.experimental.pallas import tpu_sc as plsc

B, Q, D = 2, 1000, 256
H = 8
LK = 16
Cv = 256
Hs, Ws = 100, 100
SCALE = 0.5

BQ = B * Q
BLK = 200
GRID = BQ // BLK
BPB = Q // BLK

R = H * BQ
NW = 32
RPW = R // NW
OG = 20

QS = 2.0 ** -9
QBIAS = 32768
_PERM = np.empty(Cv, dtype=np.int32)
for _t in range(Cv // 32):
    for _i in range(16):
        _PERM[32 * _t + _i] = 32 * _t + 2 * _i
        _PERM[32 * _t + 16 + _i] = 32 * _t + 2 * _i + 1
KR = 5
CH = 100


def _sigmoid(x):
    return 1.0 / (1.0 + jnp.exp(-x))


def _stage_a_body(q_ref, g_ref, wa_ref, ba_ref, wo_ref, bo_ref,
                  idx_ref, w_ref, sumw_ref):
    i = pl.program_id(0)
    base = (i // BPB) * (Hs * Ws)

    q = q_ref[...]
    geom = g_ref[...]
    box_x = _sigmoid(geom[:, 0:1])
    box_y = _sigmoid(geom[:, 1:2])
    sx = box_x * (SCALE / LK)
    sy = box_y * (SCALE / LK)

    sumw_cols = []
    for h in range(H):
        logits = jnp.dot(q, wa_ref[h], preferred_element_type=jnp.float32)
        logits = logits + ba_ref[h][None, :]
        m = jnp.max(logits, axis=1, keepdims=True)
        e = jnp.exp(logits - m)
        attn = e / jnp.sum(e, axis=1, keepdims=True)

        off = jnp.dot(q, wo_ref[h], preferred_element_type=jnp.float32)
        off = off + bo_ref[h][None, :]
        locx = box_x + off[:, :LK] * sx
        locy = box_y + off[:, LK:] * sy

        gnx = jnp.clip(2.0 * locx - 1.0, -1.0, 1.0)
        gny = jnp.clip(2.0 * locy - 1.0, -1.0, 1.0)
        gx = ((gnx + 1.0) * Ws - 1.0) * 0.5
        gy = ((gny + 1.0) * Hs - 1.0) * 0.5

        x0f = jnp.floor(gx)
        y0f = jnp.floor(gy)
        wx1 = gx - x0f
        wx0 = 1.0 - wx1
        wy1 = gy - y0f
        wy0 = 1.0 - wy1
        x1f = x0f + 1.0
        y1f = y0f + 1.0

        def corner(xf, yf, wx, wy):
            valid = ((xf >= 0.0) & (xf <= Ws - 1.0)
                     & (yf >= 0.0) & (yf <= Hs - 1.0))
            xi = jnp.clip(xf, 0.0, Ws - 1.0).astype(jnp.int32)
            yi = jnp.clip(yf, 0.0, Hs - 1.0).astype(jnp.int32)
            idx = yi * Ws + xi + base
            w = attn * (wx * wy) * valid.astype(jnp.float32)
            return idx, w

        i00, w00 = corner(x0f, y0f, wx0, wy0)
        i10, w10 = corner(x1f, y0f, wx1, wy0)
        i01, w01 = corner(x0f, y1f, wx0, wy1)
        i11, w11 = corner(x1f, y1f, wx1, wy1)

        idx64 = jnp.concatenate([i00, i10, i01, i11], axis=1)
        w64 = jnp.concatenate([w00, w10, w01, w11], axis=1)
        idx_ref[h] = idx64
        w_ref[h] = w64 * QS
        sumw_cols.append(jnp.sum(w64, axis=1, keepdims=True))

    sumw_ref[...] = jnp.concatenate(sumw_cols, axis=1)


def _run_stage_a(q2d, geom, wa, ba, wo, bo):
    return pl.pallas_call(
        _stage_a_body,
        grid=(GRID,),
        in_specs=[
            pl.BlockSpec((BLK, D), lambda i: (i, 0)),
            pl.BlockSpec((BLK, 4), lambda i: (i, 0)),
            pl.BlockSpec((H, D, LK), lambda i: (0, 0, 0)),
            pl.BlockSpec((H, LK), lambda i: (0, 0)),
            pl.BlockSpec((H, D, 2 * LK), lambda i: (0, 0, 0)),
            pl.BlockSpec((H, 2 * LK), lambda i: (0, 0)),
        ],
        out_specs=[
            pl.BlockSpec((H, BLK, 64), lambda i: (0, i, 0)),
            pl.BlockSpec((H, BLK, 64), lambda i: (0, i, 0)),
            pl.BlockSpec((BLK, H), lambda i: (i, 0)),
        ],
        out_shape=[
            jax.ShapeDtypeStruct((H, BQ, 64), jnp.int32),
            jax.ShapeDtypeStruct((H, BQ, 64), jnp.float32),
            jax.ShapeDtypeStruct((BQ, H), jnp.float32),
        ],
    )(q2d, geom, wa, ba, wo, bo)


def _sc_body(table_hbm, idx_hbm, w_hbm, out_hbm, idx_c, w_c, gbuf, obuf, gsem):
    wid = lax.axis_index("s") * 2 + lax.axis_index("c")
    rbase = wid * RPW

    pltpu.sync_copy(idx_hbm.at[pl.ds(rbase * 64, CH * 64)],
                    idx_c.at[pl.ds(0, CH * 64)])
    pltpu.sync_copy(w_hbm.at[pl.ds(rbase * 64, CH * 64)],
                    w_c.at[pl.ds(0, CH * 64)])
    for k in range(KR - 1):
        pltpu.async_copy(table_hbm.at[idx_c.at[pl.ds(k * 64, 64)]],
                         gbuf.at[k], gsem.at[k])

    def step(i, carry):
        p = lax.rem(i, KR)
        cslot = lax.rem(i // CH, 2)
        nxt = i + KR - 1

        @pl.when(jnp.logical_and(nxt < RPW, lax.rem(nxt, CH) == 0))
        def _():
            nbase = lax.rem(nxt // CH, 2) * (CH * 64)
            off = (rbase + nxt) * 64
            pltpu.sync_copy(idx_hbm.at[pl.ds(off, CH * 64)],
                            idx_c.at[pl.ds(nbase, CH * 64)])
            pltpu.sync_copy(w_hbm.at[pl.ds(off, CH * 64)],
                            w_c.at[pl.ds(nbase, CH * 64)])

        @pl.when(nxt < RPW)
        def _():
            nbase = lax.rem(nxt // CH, 2) * (CH * 64)
            loff = nbase + lax.rem(nxt, CH) * 64
            pltpu.async_copy(table_hbm.at[idx_c.at[pl.ds(loff, 64)]],
                             gbuf.at[lax.rem(nxt, KR)],
                             gsem.at[lax.rem(nxt, KR)])

        pltpu.make_async_copy(
            table_hbm.at[idx_c.at[pl.ds(cslot * (CH * 64)
                                        + lax.rem(i, CH) * 64, 64)]],
            gbuf.at[p], gsem.at[p]).wait()

        woff = cslot * (CH * 64) + lax.rem(i, CH) * 64

        def jjbody(jj, carry):
            acc, wsum = carry
            wvec = w_c[pl.ds(woff + jj * 16, 16)]
            for l in range(16):
                j = jj * 16 + l
                ws = wvec[l]
                wsum = wsum + ws
                new = list(acc)
                for t in range(8):
                    x = gbuf[p, j, pl.ds(t * 16, 16)]
                    lo = jnp.bitwise_and(x, 65535).astype(jnp.float32)
                    hi = lax.shift_right_logical(x, 16).astype(jnp.float32)
                    new[2 * t] = new[2 * t] + lo * ws
                    new[2 * t + 1] = new[2 * t + 1] + hi * ws
                acc = tuple(new)
            return acc, wsum

        acc, wsum = lax.fori_loop(
            0, 4, jjbody,
            (tuple(jnp.zeros((16,), jnp.float32) for _ in range(16)),
             jnp.zeros((16,), jnp.float32)))

        beta = wsum * float(QBIAS)
        r = lax.rem(i, OG)
        for c in range(16):
            obuf[pl.ds(r * Cv + c * 16, 16)] = acc[c] - beta

        @pl.when(r == OG - 1)
        def _():
            pltpu.sync_copy(
                obuf, out_hbm.at[pl.ds((rbase + i + 1 - OG) * Cv, OG * Cv)])

        return carry

    lax.fori_loop(0, RPW, step, 0)


def _run_stage_b(table, idx_flat, w_flat):
    mesh = plsc.VectorSubcoreMesh(core_axis_name="c", subcore_axis_name="s")
    f = functools.partial(
        pl.kernel,
        out_type=jax.ShapeDtypeStruct((R * Cv,), jnp.float32),
        mesh=mesh,
        scratch_types=[
            pltpu.VMEM((2 * CH * 64,), jnp.int32),
            pltpu.VMEM((2 * CH * 64,), jnp.float32),
            pltpu.VMEM((KR, 64, Cv // 2), jnp.int32),
            pltpu.VMEM((OG * Cv,), jnp.float32),
            pltpu.SemaphoreType.DMA((KR,)),
        ],
    )(_sc_body)
    return f(table, idx_flat, w_flat)


def _precompute_body(wv_ref, wf_ref, bv_ref, wc_ref, bb_ref):
    wv = wv_ref[...]
    bv = bv_ref[...]
    bb_rows = []
    for h in range(H):
        wf = wf_ref[h]
        wc_ref[h] = jnp.dot(wv, wf, preferred_element_type=jnp.float32)
        bb_rows.append(jnp.dot(bv, wf, preferred_element_type=jnp.float32))
    bb_ref[...] = jnp.concatenate(bb_rows, axis=0)


def _run_precompute(W_val, W_final, b_val_2d):
    return pl.pallas_call(
        _precompute_body,
        out_shape=[
            jax.ShapeDtypeStruct((H, Cv, Cv), jnp.float32),
            jax.ShapeDtypeStruct((H, Cv), jnp.float32),
        ],
    )(W_val, W_final, b_val_2d)


def _stage_c_body(heads_ref, sumw_ref, wc_ref, bb_ref, bf_ref, out_ref):
    acc = jnp.dot(sumw_ref[...], bb_ref[...],
                  preferred_element_type=jnp.float32)
    for h in range(H):
        acc = acc + jnp.dot(heads_ref[h], wc_ref[h],
                            preferred_element_type=jnp.float32)
    acc = acc + jnp.sum(bf_ref[...], axis=0)[None, :]
    out_ref[...] = acc


def _run_stage_c(heads3, sumw, wc, bb, b_final):
    return pl.pallas_call(
        _stage_c_body,
        grid=(GRID,),
        in_specs=[
            pl.BlockSpec((H, BLK, Cv), lambda i: (0, i, 0)),
            pl.BlockSpec((BLK, H), lambda i: (i, 0)),
            pl.BlockSpec((H, Cv, Cv), lambda i: (0, 0, 0)),
            pl.BlockSpec((H, Cv), lambda i: (0, 0)),
            pl.BlockSpec((H, Cv), lambda i: (0, 0)),
        ],
        out_specs=pl.BlockSpec((BLK, Cv), lambda i: (i, 0)),
        out_shape=jax.ShapeDtypeStruct((BQ, Cv), jnp.float32),
    )(heads3, sumw, wc, bb, b_final)


def kernel(queries, query_geometry_logits, value, W_off, b_off, W_attn,
           b_attn, W_val, b_val, W_final, b_final):
    q2d = queries.reshape(BQ, D)
    geom = query_geometry_logits.reshape(BQ, 4)

    wo = W_off.reshape(D, H, LK, 2).transpose(1, 0, 3, 2).reshape(H, D, 2 * LK)
    bo = b_off.reshape(H, LK, 2).transpose(0, 2, 1).reshape(H, 2 * LK)
    wa = W_attn.reshape(D, H, LK).transpose(1, 0, 2)
    ba = b_attn.reshape(H, LK)

    idx8, w8, sumw = _run_stage_a(q2d, geom, wa, ba, wo, bo)

    table = jnp.transpose(value, (0, 2, 3, 1)).reshape(B * Hs * Ws, Cv)
    tq = (jnp.clip(jnp.round(table * (1.0 / QS)), -QBIAS, QBIAS - 1)
          .astype(jnp.int32) + QBIAS).reshape(B * Hs * Ws, Cv // 2, 2)
    t_i32 = tq[:, :, 0] + (tq[:, :, 1] << 16)
    heads = _run_stage_b(t_i32, idx8.reshape(R * 64), w8.reshape(R * 64))

    wc, bb = _run_precompute(W_val[_PERM, :], W_final, b_val.reshape(1, Cv))
    out2d = _run_stage_c(heads.reshape(H, BQ, Cv), sumw, wc, bb, b_final)
    return out2d.reshape(B, Q, Cv)


# speedup vs baseline: 1.5505x; 1.0170x over previous
import functools

import numpy as np

import jax
import jax.numpy as jnp
from jax import lax
from jax.experimental import pallas as pl
from jax.experimental.pallas import tpu as pltpu
from jax.experimental.pallas import tpu_sc as plsc

B, Q, D = 2, 1000, 256
H = 8
LK = 16
Cv = 256
Hs, Ws = 100, 100
SCALE = 0.5

BQ = B * Q
BLK = 200
GRID = BQ // BLK
BPB = Q // BLK

R = H * BQ
NW = 32
RPW = R // NW
OG = 20

QS = 2.0 ** -9
QBIAS = 32768
_PERM = np.empty(Cv, dtype=np.int32)
for _t in range(Cv // 32):
    for _i in range(16):
        _PERM[32 * _t + _i] = 32 * _t + 2 * _i
        _PERM[32 * _t + 16 + _i] = 32 * _t + 2 * _i + 1
KR = 5
CH = 100


def _sigmoid(x):
    return 1.0 / (1.0 + jnp.exp(-x))


def _stage_a_body(q_ref, g_ref, wa_ref, ba_ref, wo_ref, bo_ref,
                  idx_ref, w_ref, sumw_ref):
    i = pl.program_id(0)
    base = (i // BPB) * (Hs * Ws)

    q = q_ref[...]
    geom = g_ref[...]
    box_x = _sigmoid(geom[:, 0:1])
    box_y = _sigmoid(geom[:, 1:2])
    sx = box_x * (SCALE / LK)
    sy = box_y * (SCALE / LK)

    sumw_cols = []
    for h in range(H):
        logits = jnp.dot(q, wa_ref[h], preferred_element_type=jnp.float32)
        logits = logits + ba_ref[h][None, :]
        m = jnp.max(logits, axis=1, keepdims=True)
        e = jnp.exp(logits - m)
        attn = e / jnp.sum(e, axis=1, keepdims=True)

        off = jnp.dot(q, wo_ref[h], preferred_element_type=jnp.float32)
        off = off + bo_ref[h][None, :]
        locx = box_x + off[:, :LK] * sx
        locy = box_y + off[:, LK:] * sy

        gnx = jnp.clip(2.0 * locx - 1.0, -1.0, 1.0)
        gny = jnp.clip(2.0 * locy - 1.0, -1.0, 1.0)
        gx = ((gnx + 1.0) * Ws - 1.0) * 0.5
        gy = ((gny + 1.0) * Hs - 1.0) * 0.5

        x0f = jnp.floor(gx)
        y0f = jnp.floor(gy)
        wx1 = gx - x0f
        wx0 = 1.0 - wx1
        wy1 = gy - y0f
        wy0 = 1.0 - wy1
        x1f = x0f + 1.0
        y1f = y0f + 1.0

        def corner(xf, yf, wx, wy):
            valid = ((xf >= 0.0) & (xf <= Ws - 1.0)
                     & (yf >= 0.0) & (yf <= Hs - 1.0))
            xi = jnp.clip(xf, 0.0, Ws - 1.0).astype(jnp.int32)
            yi = jnp.clip(yf, 0.0, Hs - 1.0).astype(jnp.int32)
            idx = yi * Ws + xi + base
            w = attn * (wx * wy) * valid.astype(jnp.float32)
            return idx, w

        i00, w00 = corner(x0f, y0f, wx0, wy0)
        i10, w10 = corner(x1f, y0f, wx1, wy0)
        i01, w01 = corner(x0f, y1f, wx0, wy1)
        i11, w11 = corner(x1f, y1f, wx1, wy1)

        idx64 = jnp.concatenate([i00, i10, i01, i11], axis=1)
        w64 = jnp.concatenate([w00, w10, w01, w11], axis=1)
        idx_ref[h] = idx64
        w_ref[h] = w64 * QS
        sumw_cols.append(jnp.sum(w64, axis=1, keepdims=True))

    sumw_ref[...] = jnp.concatenate(sumw_cols, axis=1)


def _run_stage_a(q2d, geom, wa, ba, wo, bo):
    return pl.pallas_call(
        _stage_a_body,
        grid=(GRID,),
        in_specs=[
            pl.BlockSpec((BLK, D), lambda i: (i, 0)),
            pl.BlockSpec((BLK, 4), lambda i: (i, 0)),
            pl.BlockSpec((H, D, LK), lambda i: (0, 0, 0)),
            pl.BlockSpec((H, LK), lambda i: (0, 0)),
            pl.BlockSpec((H, D, 2 * LK), lambda i: (0, 0, 0)),
            pl.BlockSpec((H, 2 * LK), lambda i: (0, 0)),
        ],
        out_specs=[
            pl.BlockSpec((H, BLK, 64), lambda i: (0, i, 0)),
            pl.BlockSpec((H, BLK, 64), lambda i: (0, i, 0)),
            pl.BlockSpec((BLK, H), lambda i: (i, 0)),
        ],
        out_shape=[
            jax.ShapeDtypeStruct((H, BQ, 64), jnp.int32),
            jax.ShapeDtypeStruct((H, BQ, 64), jnp.float32),
            jax.ShapeDtypeStruct((BQ, H), jnp.float32),
        ],
    )(q2d, geom, wa, ba, wo, bo)


def _sc_body(table_hbm, idx_hbm, w_hbm, out_hbm, idx_c, w_c, gbuf, obuf, gsem,
             csem, osem):
    wid = lax.axis_index("s") * 2 + lax.axis_index("c")
    rbase = wid * RPW

    pltpu.sync_copy(idx_hbm.at[pl.ds(rbase * 64, CH * 64)],
                    idx_c.at[pl.ds(0, CH * 64)])
    pltpu.sync_copy(w_hbm.at[pl.ds(rbase * 64, CH * 64)],
                    w_c.at[pl.ds(0, CH * 64)])
    for k in range(KR - 1):
        pltpu.async_copy(table_hbm.at[idx_c.at[pl.ds(k * 64, 64)]],
                         gbuf.at[k], gsem.at[k])

    def step(i, carry):
        p = lax.rem(i, KR)
        cslot = lax.rem(i // CH, 2)
        nxt = i + KR - 1

        nchunk = nxt // CH + 1

        @pl.when(jnp.logical_and(nchunk * CH < RPW,
                                 lax.rem(nxt, CH) == CH // 2))
        def _():
            nbase = lax.rem(nchunk, 2) * (CH * 64)
            off = (rbase + nchunk * CH) * 64
            pltpu.async_copy(idx_hbm.at[pl.ds(off, CH * 64)],
                             idx_c.at[pl.ds(nbase, CH * 64)], csem.at[0])
            pltpu.async_copy(w_hbm.at[pl.ds(off, CH * 64)],
                             w_c.at[pl.ds(nbase, CH * 64)], csem.at[1])

        @pl.when(jnp.logical_and(nxt < RPW, lax.rem(nxt, CH) == 0))
        def _():
            nbase = lax.rem(nxt // CH, 2) * (CH * 64)
            off = (rbase + nxt) * 64
            pltpu.make_async_copy(idx_hbm.at[pl.ds(off, CH * 64)],
                                  idx_c.at[pl.ds(nbase, CH * 64)],
                                  csem.at[0]).wait()
            pltpu.make_async_copy(w_hbm.at[pl.ds(off, CH * 64)],
                                  w_c.at[pl.ds(nbase, CH * 64)],
                                  csem.at[1]).wait()

        @pl.when(nxt < RPW)
        def _():
            nbase = lax.rem(nxt // CH, 2) * (CH * 64)
            loff = nbase + lax.rem(nxt, CH) * 64
            pltpu.async_copy(table_hbm.at[idx_c.at[pl.ds(loff, 64)]],
                             gbuf.at[lax.rem(nxt, KR)],
                             gsem.at[lax.rem(nxt, KR)])

        pltpu.make_async_copy(
            table_hbm.at[idx_c.at[pl.ds(cslot * (CH * 64)
                                        + lax.rem(i, CH) * 64, 64)]],
            gbuf.at[p], gsem.at[p]).wait()

        woff = cslot * (CH * 64) + lax.rem(i, CH) * 64

        def jjbody(jj, carry):
            acc, wsum = carry
            wvec = w_c[pl.ds(woff + jj * 16, 16)]
            for l in range(16):
                j = jj * 16 + l
                ws = wvec[l]
                wsum = wsum + ws
                new = list(acc)
                for t in range(8):
                    x = gbuf[p, j, pl.ds(t * 16, 16)]
                    lo = jnp.bitwise_and(x, 65535).astype(jnp.float32)
                    hi = lax.shift_right_logical(x, 16).astype(jnp.float32)
                    new[2 * t] = new[2 * t] + lo * ws
                    new[2 * t + 1] = new[2 * t + 1] + hi * ws
                acc = tuple(new)
            return acc, wsum

        acc, wsum = lax.fori_loop(
            0, 4, jjbody,
            (tuple(jnp.zeros((16,), jnp.float32) for _ in range(16)),
             jnp.zeros((16,), jnp.float32)))

        beta = wsum * float(QBIAS)
        r = lax.rem(i, OG)
        oslot = lax.rem(i // OG, 2) * (OG * Cv)

        @pl.when(jnp.logical_and(r == 0, i >= 2 * OG))
        def _():
            pltpu.make_async_copy(
                obuf.at[pl.ds(oslot, OG * Cv)],
                out_hbm.at[pl.ds((rbase + i - 2 * OG) * Cv, OG * Cv)],
                osem.at[lax.rem(i // OG, 2)]).wait()

        for c in range(16):
            obuf[pl.ds(oslot + r * Cv + c * 16, 16)] = acc[c] - beta

        @pl.when(r == OG - 1)
        def _():
            pltpu.async_copy(
                obuf.at[pl.ds(oslot, OG * Cv)],
                out_hbm.at[pl.ds((rbase + i + 1 - OG) * Cv, OG * Cv)],
                osem.at[lax.rem(i // OG, 2)])

        return carry

    lax.fori_loop(0, RPW, step, 0)
    for g in (RPW // OG - 2, RPW // OG - 1):
        pltpu.make_async_copy(
            obuf.at[pl.ds((g % 2) * (OG * Cv), OG * Cv)],
            out_hbm.at[pl.ds((rbase + g * OG) * Cv, OG * Cv)],
            osem.at[g % 2]).wait()


def _run_stage_b(table, idx_flat, w_flat):
    mesh = plsc.VectorSubcoreMesh(core_axis_name="c", subcore_axis_name="s")
    f = functools.partial(
        pl.kernel,
        out_type=jax.ShapeDtypeStruct((R * Cv,), jnp.float32),
        mesh=mesh,
        scratch_types=[
            pltpu.VMEM((2 * CH * 64,), jnp.int32),
            pltpu.VMEM((2 * CH * 64,), jnp.float32),
            pltpu.VMEM((KR, 64, Cv // 2), jnp.int32),
            pltpu.VMEM((2 * OG * Cv,), jnp.float32),
            pltpu.SemaphoreType.DMA((KR,)),
            pltpu.SemaphoreType.DMA((2,)),
            pltpu.SemaphoreType.DMA((2,)),
        ],
    )(_sc_body)
    return f(table, idx_flat, w_flat)


def _precompute_body(wv_ref, wf_ref, bv_ref, wc_ref, bb_ref):
    wv = wv_ref[...]
    bv = bv_ref[...]
    bb_rows = []
    for h in range(H):
        wf = wf_ref[h]
        wc_ref[h] = jnp.dot(wv, wf, preferred_element_type=jnp.float32)
        bb_rows.append(jnp.dot(bv, wf, preferred_element_type=jnp.float32))
    bb_ref[...] = jnp.concatenate(bb_rows, axis=0)


def _run_precompute(W_val, W_final, b_val_2d):
    return pl.pallas_call(
        _precompute_body,
        out_shape=[
            jax.ShapeDtypeStruct((H, Cv, Cv), jnp.float32),
            jax.ShapeDtypeStruct((H, Cv), jnp.float32),
        ],
    )(W_val, W_final, b_val_2d)


def _stage_c_body(heads_ref, sumw_ref, wc_ref, bb_ref, bf_ref, out_ref):
    acc = jnp.dot(sumw_ref[...], bb_ref[...],
                  preferred_element_type=jnp.float32)
    for h in range(H):
        acc = acc + jnp.dot(heads_ref[h], wc_ref[h],
                            preferred_element_type=jnp.float32)
    acc = acc + jnp.sum(bf_ref[...], axis=0)[None, :]
    out_ref[...] = acc


def _run_stage_c(heads3, sumw, wc, bb, b_final):
    return pl.pallas_call(
        _stage_c_body,
        grid=(GRID,),
        in_specs=[
            pl.BlockSpec((H, BLK, Cv), lambda i: (0, i, 0)),
            pl.BlockSpec((BLK, H), lambda i: (i, 0)),
            pl.BlockSpec((H, Cv, Cv), lambda i: (0, 0, 0)),
            pl.BlockSpec((H, Cv), lambda i: (0, 0)),
            pl.BlockSpec((H, Cv), lambda i: (0, 0)),
        ],
        out_specs=pl.BlockSpec((BLK, Cv), lambda i: (i, 0)),
        out_shape=jax.ShapeDtypeStruct((BQ, Cv), jnp.float32),
    )(heads3, sumw, wc, bb, b_final)


def kernel(queries, query_geometry_logits, value, W_off, b_off, W_attn,
           b_attn, W_val, b_val, W_final, b_final):
    q2d = queries.reshape(BQ, D)
    geom = query_geometry_logits.reshape(BQ, 4)

    wo = W_off.reshape(D, H, LK, 2).transpose(1, 0, 3, 2).reshape(H, D, 2 * LK)
    bo = b_off.reshape(H, LK, 2).transpose(0, 2, 1).reshape(H, 2 * LK)
    wa = W_attn.reshape(D, H, LK).transpose(1, 0, 2)
    ba = b_attn.reshape(H, LK)

    idx8, w8, sumw = _run_stage_a(q2d, geom, wa, ba, wo, bo)

    table = jnp.transpose(value, (0, 2, 3, 1)).reshape(B * Hs * Ws, Cv)
    tq = (jnp.clip(jnp.round(table * (1.0 / QS)), -QBIAS, QBIAS - 1)
          .astype(jnp.int32) + QBIAS).reshape(B * Hs * Ws, Cv // 2, 2)
    t_i32 = tq[:, :, 0] + (tq[:, :, 1] << 16)
    heads = _run_stage_b(t_i32, idx8.reshape(R * 64), w8.reshape(R * 64))

    wc, bb = _run_precompute(W_val[_PERM, :], W_final, b_val.reshape(1, Cv))
    out2d = _run_stage_c(heads.reshape(H, BQ, Cv), sumw, wc, bb, b_final)
    return out2d.reshape(B, Q, Cv)


# speedup vs baseline: 1.6620x; 1.0719x over previous
import functools

import numpy as np

import jax
import jax.numpy as jnp
from jax import lax
from jax.experimental import pallas as pl
from jax.experimental.pallas import tpu as pltpu
from jax.experimental.pallas import tpu_sc as plsc

B, Q, D = 2, 1000, 256
H = 8
LK = 16
Cv = 256
Hs, Ws = 100, 100
SCALE = 0.5

BQ = B * Q
BLK = 200
GRID = BQ // BLK
BPB = Q // BLK

R = H * BQ
NW = 32
RPW = R // NW
OG = 20

HLK = H * LK
_SEG = np.kron(np.eye(H * LK // LK, dtype=np.float32),
               np.ones((LK, LK), np.float32))

QS = 2.0 ** -9
QBIAS = 32768
_PERM = np.empty(Cv, dtype=np.int32)
for _t in range(Cv // 32):
    for _i in range(16):
        _PERM[32 * _t + _i] = 32 * _t + 2 * _i
        _PERM[32 * _t + 16 + _i] = 32 * _t + 2 * _i + 1
KR = 5
CH = 100


def _sigmoid(x):
    return 1.0 / (1.0 + jnp.exp(-x))


def _stage_a_body(q_ref, g_ref, wa_ref, ba_ref, wo_ref, bo_ref, seg_ref,
                  idx_ref, w_ref, sumw_ref):
    i = pl.program_id(0)
    base = (i // BPB) * (Hs * Ws)

    q = q_ref[...]
    geom = g_ref[...]
    box_x = _sigmoid(geom[:, 0:1])
    box_y = _sigmoid(geom[:, 1:2])
    sx = box_x * (SCALE / LK)
    sy = box_y * (SCALE / LK)

    seg = seg_ref[...]

    logits = jnp.dot(q, wa_ref[...], preferred_element_type=jnp.float32)
    logits = logits + ba_ref[...]
    e = jnp.exp(logits - jnp.max(logits, axis=1, keepdims=True))
    attn = e / jnp.dot(e, seg, preferred_element_type=jnp.float32)

    off = jnp.dot(q, wo_ref[...], preferred_element_type=jnp.float32)
    off = off + bo_ref[...]
    locx = box_x + off[:, :HLK] * sx
    locy = box_y + off[:, HLK:] * sy

    gx = ((jnp.clip(2.0 * locx - 1.0, -1.0, 1.0) + 1.0) * Ws - 1.0) * 0.5
    gy = ((jnp.clip(2.0 * locy - 1.0, -1.0, 1.0) + 1.0) * Hs - 1.0) * 0.5

    x0f = jnp.floor(gx)
    y0f = jnp.floor(gy)
    wx1 = gx - x0f
    wx0 = 1.0 - wx1
    wy1 = gy - y0f
    wy0 = 1.0 - wy1

    def corner(xf, yf, wx, wy):
        valid = ((xf >= 0.0) & (xf <= Ws - 1.0)
                 & (yf >= 0.0) & (yf <= Hs - 1.0))
        xi = jnp.clip(xf, 0.0, Ws - 1.0).astype(jnp.int32)
        yi = jnp.clip(yf, 0.0, Hs - 1.0).astype(jnp.int32)
        idx = yi * Ws + xi + base
        w = attn * (wx * wy) * valid.astype(jnp.float32)
        return idx, w

    i00, w00 = corner(x0f, y0f, wx0, wy0)
    i10, w10 = corner(x0f + 1.0, y0f, wx1, wy0)
    i01, w01 = corner(x0f, y0f + 1.0, wx0, wy1)
    i11, w11 = corner(x0f + 1.0, y0f + 1.0, wx1, wy1)

    gsum = jnp.dot(w00 + w10 + w01 + w11, seg,
                   preferred_element_type=jnp.float32)

    zi = jnp.zeros((BLK, 64), jnp.int32)
    zf = jnp.zeros((BLK, 64), jnp.float32)
    sumw_cols = []
    for h in range(H):
        s0, s1 = 16 * h, 16 * h + 16
        idx_ref[h] = jnp.concatenate(
            [i00[:, s0:s1], i10[:, s0:s1], i01[:, s0:s1], i11[:, s0:s1], zi],
            axis=1)
        w_ref[h] = jnp.concatenate(
            [w00[:, s0:s1], w10[:, s0:s1], w01[:, s0:s1], w11[:, s0:s1], zf],
            axis=1) * QS
        sumw_cols.append(gsum[:, s0:s0 + 1])

    sumw_ref[...] = jnp.concatenate(sumw_cols, axis=1)


def _run_stage_a(q2d, geom, wa, ba, wo, bo, seg):
    return pl.pallas_call(
        _stage_a_body,
        grid=(GRID,),
        in_specs=[
            pl.BlockSpec((BLK, D), lambda i: (i, 0)),
            pl.BlockSpec((BLK, 4), lambda i: (i, 0)),
            pl.BlockSpec((D, HLK), lambda i: (0, 0)),
            pl.BlockSpec((1, HLK), lambda i: (0, 0)),
            pl.BlockSpec((D, 2 * HLK), lambda i: (0, 0)),
            pl.BlockSpec((1, 2 * HLK), lambda i: (0, 0)),
            pl.BlockSpec((HLK, HLK), lambda i: (0, 0)),
        ],
        out_specs=[
            pl.BlockSpec((H, BLK, 128), lambda i: (0, i, 0)),
            pl.BlockSpec((H, BLK, 128), lambda i: (0, i, 0)),
            pl.BlockSpec((BLK, H), lambda i: (i, 0)),
        ],
        out_shape=[
            jax.ShapeDtypeStruct((H, BQ, 128), jnp.int32),
            jax.ShapeDtypeStruct((H, BQ, 128), jnp.float32),
            jax.ShapeDtypeStruct((BQ, H), jnp.float32),
        ],
    )(q2d, geom, wa, ba, wo, bo, seg)


def _sc_body(table_hbm, idx_hbm, w_hbm, out_hbm, idx_c, w_c, gbuf, obuf, gsem,
             csem, osem):
    wid = lax.axis_index("s") * 2 + lax.axis_index("c")
    rbase = wid * RPW

    pltpu.sync_copy(idx_hbm.at[pl.ds(rbase * 128, CH * 128)],
                    idx_c.at[pl.ds(0, CH * 128)])
    pltpu.sync_copy(w_hbm.at[pl.ds(rbase * 128, CH * 128)],
                    w_c.at[pl.ds(0, CH * 128)])
    for k in range(KR - 1):
        pltpu.async_copy(table_hbm.at[idx_c.at[pl.ds(k * 128, 64)]],
                         gbuf.at[k], gsem.at[k])

    def step(i, carry):
        p = lax.rem(i, KR)
        cslot = lax.rem(i // CH, 2)
        nxt = i + KR - 1

        nchunk = nxt // CH + 1

        @pl.when(jnp.logical_and(nchunk * CH < RPW,
                                 lax.rem(nxt, CH) == CH // 2))
        def _():
            nbase = lax.rem(nchunk, 2) * (CH * 128)
            off = (rbase + nchunk * CH) * 128
            pltpu.async_copy(idx_hbm.at[pl.ds(off, CH * 128)],
                             idx_c.at[pl.ds(nbase, CH * 128)], csem.at[0])
            pltpu.async_copy(w_hbm.at[pl.ds(off, CH * 128)],
                             w_c.at[pl.ds(nbase, CH * 128)], csem.at[1])

        @pl.when(jnp.logical_and(nxt < RPW, lax.rem(nxt, CH) == 0))
        def _():
            nbase = lax.rem(nxt // CH, 2) * (CH * 128)
            off = (rbase + nxt) * 128
            pltpu.make_async_copy(idx_hbm.at[pl.ds(off, CH * 128)],
                                  idx_c.at[pl.ds(nbase, CH * 128)],
                                  csem.at[0]).wait()
            pltpu.make_async_copy(w_hbm.at[pl.ds(off, CH * 128)],
                                  w_c.at[pl.ds(nbase, CH * 128)],
                                  csem.at[1]).wait()

        @pl.when(nxt < RPW)
        def _():
            nbase = lax.rem(nxt // CH, 2) * (CH * 128)
            loff = nbase + lax.rem(nxt, CH) * 128
            pltpu.async_copy(table_hbm.at[idx_c.at[pl.ds(loff, 64)]],
                             gbuf.at[lax.rem(nxt, KR)],
                             gsem.at[lax.rem(nxt, KR)])

        pltpu.make_async_copy(
            table_hbm.at[idx_c.at[pl.ds(cslot * (CH * 128)
                                        + lax.rem(i, CH) * 128, 64)]],
            gbuf.at[p], gsem.at[p]).wait()

        woff = cslot * (CH * 128) + lax.rem(i, CH) * 128

        def jjbody(jj, carry):
            acc, wsum = carry
            wvec = w_c[pl.ds(woff + jj * 16, 16)]
            for l in range(16):
                j = jj * 16 + l
                ws = wvec[l]
                wsum = wsum + ws
                new = list(acc)
                for t in range(8):
                    x = gbuf[p, j, pl.ds(t * 16, 16)]
                    lo = jnp.bitwise_and(x, 65535).astype(jnp.float32)
                    hi = lax.shift_right_logical(x, 16).astype(jnp.float32)
                    new[2 * t] = new[2 * t] + lo * ws
                    new[2 * t + 1] = new[2 * t + 1] + hi * ws
                acc = tuple(new)
            return acc, wsum

        acc, wsum = lax.fori_loop(
            0, 4, jjbody,
            (tuple(jnp.zeros((16,), jnp.float32) for _ in range(16)),
             jnp.zeros((16,), jnp.float32)))

        beta = wsum * float(QBIAS)
        r = lax.rem(i, OG)
        oslot = lax.rem(i // OG, 2) * (OG * Cv)

        @pl.when(jnp.logical_and(r == 0, i >= 2 * OG))
        def _():
            pltpu.make_async_copy(
                obuf.at[pl.ds(oslot, OG * Cv)],
                out_hbm.at[pl.ds((rbase + i - 2 * OG) * Cv, OG * Cv)],
                osem.at[lax.rem(i // OG, 2)]).wait()

        for c in range(16):
            obuf[pl.ds(oslot + r * Cv + c * 16, 16)] = acc[c] - beta

        @pl.when(r == OG - 1)
        def _():
            pltpu.async_copy(
                obuf.at[pl.ds(oslot, OG * Cv)],
                out_hbm.at[pl.ds((rbase + i + 1 - OG) * Cv, OG * Cv)],
                osem.at[lax.rem(i // OG, 2)])

        return carry

    lax.fori_loop(0, RPW, step, 0)
    for g in (RPW // OG - 2, RPW // OG - 1):
        pltpu.make_async_copy(
            obuf.at[pl.ds((g % 2) * (OG * Cv), OG * Cv)],
            out_hbm.at[pl.ds((rbase + g * OG) * Cv, OG * Cv)],
            osem.at[g % 2]).wait()


def _run_stage_b(table, idx_flat, w_flat):
    mesh = plsc.VectorSubcoreMesh(core_axis_name="c", subcore_axis_name="s")
    f = functools.partial(
        pl.kernel,
        out_type=jax.ShapeDtypeStruct((R * Cv,), jnp.float32),
        mesh=mesh,
        scratch_types=[
            pltpu.VMEM((2 * CH * 128,), jnp.int32),
            pltpu.VMEM((2 * CH * 128,), jnp.float32),
            pltpu.VMEM((KR, 64, Cv // 2), jnp.int32),
            pltpu.VMEM((2 * OG * Cv,), jnp.float32),
            pltpu.SemaphoreType.DMA((KR,)),
            pltpu.SemaphoreType.DMA((2,)),
            pltpu.SemaphoreType.DMA((2,)),
        ],
    )(_sc_body)
    return f(table, idx_flat, w_flat)


def _precompute_body(wv_ref, wf_ref, bv_ref, wc_ref, bb_ref):
    wv = wv_ref[...]
    bv = bv_ref[...]
    bb_rows = []
    for h in range(H):
        wf = wf_ref[h]
        wc_ref[h] = jnp.dot(wv, wf, preferred_element_type=jnp.float32)
        bb_rows.append(jnp.dot(bv, wf, preferred_element_type=jnp.float32))
    bb_ref[...] = jnp.concatenate(bb_rows, axis=0)


def _run_precompute(W_val, W_final, b_val_2d):
    return pl.pallas_call(
        _precompute_body,
        out_shape=[
            jax.ShapeDtypeStruct((H, Cv, Cv), jnp.float32),
            jax.ShapeDtypeStruct((H, Cv), jnp.float32),
        ],
    )(W_val, W_final, b_val_2d)


def _stage_c_body(heads_ref, sumw_ref, wc_ref, bb_ref, bf_ref, out_ref):
    acc = jnp.dot(sumw_ref[...], bb_ref[...],
                  preferred_element_type=jnp.float32)
    for h in range(H):
        acc = acc + jnp.dot(heads_ref[h], wc_ref[h],
                            preferred_element_type=jnp.float32)
    acc = acc + jnp.sum(bf_ref[...], axis=0)[None, :]
    out_ref[...] = acc


def _run_stage_c(heads3, sumw, wc, bb, b_final):
    return pl.pallas_call(
        _stage_c_body,
        grid=(GRID,),
        in_specs=[
            pl.BlockSpec((H, BLK, Cv), lambda i: (0, i, 0)),
            pl.BlockSpec((BLK, H), lambda i: (i, 0)),
            pl.BlockSpec((H, Cv, Cv), lambda i: (0, 0, 0)),
            pl.BlockSpec((H, Cv), lambda i: (0, 0)),
            pl.BlockSpec((H, Cv), lambda i: (0, 0)),
        ],
        out_specs=pl.BlockSpec((BLK, Cv), lambda i: (i, 0)),
        out_shape=jax.ShapeDtypeStruct((BQ, Cv), jnp.float32),
    )(heads3, sumw, wc, bb, b_final)


def kernel(queries, query_geometry_logits, value, W_off, b_off, W_attn,
           b_attn, W_val, b_val, W_final, b_final):
    q2d = queries.reshape(BQ, D)
    geom = query_geometry_logits.reshape(BQ, 4)

    wo = W_off.reshape(D, H, LK, 2).transpose(0, 3, 1, 2).reshape(D, 2 * HLK)
    bo = b_off.reshape(1, H, LK, 2).transpose(0, 3, 1, 2).reshape(1, 2 * HLK)
    ba = b_attn.reshape(1, HLK)

    idx8, w8, sumw = _run_stage_a(q2d, geom, W_attn, ba, wo, bo,
                                  jnp.asarray(_SEG))

    table = jnp.transpose(value, (0, 2, 3, 1)).reshape(B * Hs * Ws, Cv)
    tq = (jnp.clip(jnp.round(table * (1.0 / QS)), -QBIAS, QBIAS - 1)
          .astype(jnp.int32) + QBIAS).reshape(B * Hs * Ws, Cv // 2, 2)
    t_i32 = tq[:, :, 0] + (tq[:, :, 1] << 16)
    heads = _run_stage_b(t_i32, idx8.reshape(R * 128), w8.reshape(R * 128))

    wc, bb = _run_precompute(W_val[_PERM, :], W_final, b_val.reshape(1, Cv))
    out2d = _run_stage_c(heads.reshape(H, BQ, Cv), sumw, wc, bb, b_final)
    return out2d.reshape(B, Q, Cv)
